# baseline (speedup 1.0000x reference)
import functools

import jax
import jax.numpy as jnp
from jax import lax
from jax.experimental import pallas as pl
from jax.experimental.pallas import tpu as pltpu
from jax.experimental.pallas import tpu_sc as plsc

_N = 100000
_E = 3200000
_IN = 16
_HALF = 8

_TILES = 16
_KB = 8
_CE = _KB * 128
_EROWS = _E // 128
_ROWS = 25088
_PROWS = _ROWS - _EROWS
_ROWS_PER_TILE = _ROWS // _TILES
_CHUNKS = _ROWS_PER_TILE // _KB
_NPAD = 100096
_TSLICE = _NPAD // _TILES
_FSLICE = _N // _TILES


def _sc_body(feat_a, feat_b, esrc, edst, epad, zeros_h,
             z1o, z2o, z4o, pmo, fa_tbl,
             accum, srcv, dstv, rows, fbuf, isem, gsem, ssem):
    c = lax.axis_index("c")
    s = lax.axis_index("s")
    rbase = s * _ROWS_PER_TILE

    for h in range(2):
        r = s * _FSLICE + h * (_FSLICE // 2)
        pltpu.sync_copy(
            feat_a.at[pl.ds(r, _FSLICE // 2), pl.ds(c * _HALF, _HALF)], fbuf)
        pltpu.sync_copy(fbuf, fa_tbl.at[c, pl.ds(r, _FSLICE // 2)])

    @pl.when(s == 0)
    def _():
        pltpu.sync_copy(zeros_h.at[pl.ds(_N, _NPAD - _N)],
                        fa_tbl.at[c, pl.ds(_N, _NPAD - _N)])

    def zero_slice():
        pltpu.sync_copy(zeros_h.at[pl.ds(s * _TSLICE, _TSLICE)],
                        accum.at[pl.ds(s * _TSLICE, _TSLICE)])

    def writeout(out):
        pltpu.sync_copy(accum.at[pl.ds(s * _TSLICE, _TSLICE)],
                        out.at[c, pl.ds(s * _TSLICE, _TSLICE)])

    def load_idx(r0):
        @pl.when(r0 < _EROWS)
        def _():
            c1 = pltpu.async_copy(esrc.at[pl.ds(r0, _KB)], srcv, isem)
            c2 = pltpu.async_copy(edst.at[pl.ds(r0, _KB)], dstv, isem)
            c1.wait()
            c2.wait()

        @pl.when(r0 >= _EROWS)
        def _():
            c1 = pltpu.async_copy(epad.at[0, pl.ds(r0 - _EROWS, _KB)],
                                  srcv, isem)
            c2 = pltpu.async_copy(epad.at[1, pl.ds(r0 - _EROWS, _KB)],
                                  dstv, isem)
            c1.wait()
            c2.wait()

    def agg_pass(tbl):
        def body(i, carry):
            r0 = rbase + i * _KB
            load_idx(r0)
            gs = [pltpu.async_copy(tbl.at[srcv.at[j]],
                                   rows.at[pl.ds(j * 128, 128)], gsem)
                  for j in range(_KB)]
            for g in gs:
                g.wait()
            ss = [pltpu.async_copy(rows.at[pl.ds(j * 128, 128)],
                                   accum.at[dstv.at[j]], ssem, add=True)
                  for j in range(_KB)]
            for t in ss:
                t.wait()
            return carry
        lax.fori_loop(0, _CHUNKS, body, 0)

    def pmpd_pass():
        def body(i, carry):
            r0 = rbase + i * _KB
            load_idx(r0)

            @pl.when(r0 < _EROWS)
            def _():
                pltpu.async_copy(
                    feat_b.at[pl.ds(r0 * 128, _CE), pl.ds(c * _HALF, _HALF)],
                    rows, gsem).wait()

            ss = []
            for j in range(_KB):
                ss.append(pltpu.async_copy(rows.at[pl.ds(j * 128, 128)],
                                           accum.at[srcv.at[j]], ssem,
                                           add=True))
                ss.append(pltpu.async_copy(rows.at[pl.ds(j * 128, 128)],
                                           accum.at[dstv.at[j]], ssem,
                                           add=True))
            for t in ss:
                t.wait()
            return carry
        lax.fori_loop(0, _CHUNKS, body, 0)

    zero_slice()
    plsc.subcore_barrier()
    agg_pass(fa_tbl.at[c])
    plsc.subcore_barrier()
    writeout(z1o)
    zero_slice()
    plsc.subcore_barrier()
    agg_pass(z1o.at[c])
    plsc.subcore_barrier()
    writeout(z2o)
    zero_slice()
    plsc.subcore_barrier()
    agg_pass(z2o.at[c])
    plsc.subcore_barrier()
    writeout(z4o)
    zero_slice()
    plsc.subcore_barrier()
    agg_pass(z4o.at[c])
    plsc.subcore_barrier()
    writeout(z4o)
    zero_slice()
    plsc.subcore_barrier()
    pmpd_pass()
    plsc.subcore_barrier()
    writeout(pmo)


_sc_call = functools.partial(
    pl.kernel,
    out_type=[jax.ShapeDtypeStruct((2, _NPAD, _HALF), jnp.float32)] * 5,
    mesh=plsc.VectorSubcoreMesh(core_axis_name="c", subcore_axis_name="s"),
    scratch_types=[
        pltpu.VMEM_SHARED((_NPAD, _HALF), jnp.float32),
        pltpu.VMEM((_KB, 128), jnp.int32),
        pltpu.VMEM((_KB, 128), jnp.int32),
        pltpu.VMEM((_CE, _HALF), jnp.float32),
        pltpu.VMEM((_FSLICE // 2, _HALF), jnp.float32),
        pltpu.SemaphoreType.DMA,
        pltpu.SemaphoreType.DMA,
        pltpu.SemaphoreType.DMA,
    ],
    compiler_params=pltpu.CompilerParams(use_tc_tiling_on_sc=False),
)(_sc_body)


_BN = 2000
_GRID = _N // _BN


def _fuse_body(fa, dg, z1, z2, z4, pm, wp, wd, wr, bsum, res, sums):
    x = fa[...]
    w = wr[...]
    acc = jnp.dot(x, wp[...], preferred_element_type=jnp.float32)
    acc += jnp.dot(dg[...] * x, wd[...], preferred_element_type=jnp.float32)
    acc += jnp.dot(z1[0], w[0], preferred_element_type=jnp.float32)
    acc += jnp.dot(z1[1], w[1], preferred_element_type=jnp.float32)
    acc += jnp.dot(z2[0], w[2], preferred_element_type=jnp.float32)
    acc += jnp.dot(z2[1], w[3], preferred_element_type=jnp.float32)
    acc += jnp.dot(z4[0], w[4], preferred_element_type=jnp.float32)
    acc += jnp.dot(z4[1], w[5], preferred_element_type=jnp.float32)
    acc += jnp.dot(pm[0], w[6], preferred_element_type=jnp.float32)
    acc += jnp.dot(pm[1], w[7], preferred_element_type=jnp.float32)
    acc += bsum[...]
    col = lax.broadcasted_iota(jnp.int32, acc.shape, 1)
    acc = jnp.where((col >= _IN // 2) & (acc < 0.0), 0.0, acc)
    res[...] = acc
    sums[...] = jnp.stack([jnp.sum(acc, axis=0),
                           jnp.sum(acc * acc, axis=0)])[None]


def _bn_body(res, scale, shift, out):
    out[...] = res[...] * scale[...] + shift[...]


def kernel(feat_a, feat_b, deg, edge_index, Wp, bp, Wd, bd, Wr0, br0,
           Wr1, br1, Wr2, br2, Wf, bf, gamma, beta):
    f32 = jnp.float32
    esrc = edge_index[0].reshape(_EROWS, 128)
    edst = edge_index[1].reshape(_EROWS, 128)
    epad = jnp.full((2, _PROWS, 128), _N, jnp.int32)
    zeros_h = jnp.zeros((_NPAD, _HALF), f32)

    z1h, z2h, z4h, pmh, _ = _sc_call(feat_a, feat_b, esrc, edst, epad,
                                     zeros_h)

    wr = jnp.stack([Wr0[:_HALF], Wr0[_HALF:], Wr1[:_HALF], Wr1[_HALF:],
                    Wr2[:_HALF], Wr2[_HALF:], Wf[:_HALF], Wf[_HALF:]])
    bsum = (bp + bd + br0 + br1 + br2 + bf).reshape(1, _IN)

    row_spec = pl.BlockSpec((_BN, _IN), lambda i: (i, 0))
    half_spec = pl.BlockSpec((2, _BN, _HALF), lambda i: (0, i, 0))
    deg_spec = pl.BlockSpec((_BN, 1), lambda i: (i, 0))
    w_spec = pl.BlockSpec((_IN, _IN), lambda i: (0, 0))
    wr_spec = pl.BlockSpec((8, _HALF, _IN), lambda i: (0, 0, 0))
    v_spec = pl.BlockSpec((1, _IN), lambda i: (0, 0))

    res, sums = pl.pallas_call(
        _fuse_body,
        grid=(_GRID,),
        in_specs=[row_spec, deg_spec, half_spec, half_spec, half_spec,
                  half_spec, w_spec, w_spec, wr_spec, v_spec],
        out_specs=[row_spec, pl.BlockSpec((1, 2, _IN), lambda i: (i, 0, 0))],
        out_shape=[jax.ShapeDtypeStruct((_N, _IN), f32),
                   jax.ShapeDtypeStruct((_GRID, 2, _IN), f32)],
    )(feat_a, deg, z1h, z2h, z4h, pmh, Wp, Wd, wr, bsum)

    st = jnp.sum(sums, axis=0)
    mean = st[0] / _N
    var = st[1] / _N - mean * mean
    rstd = lax.rsqrt(var + 1e-5)
    scale = (gamma * rstd).reshape(1, _IN)
    shift = (beta - mean * gamma * rstd).reshape(1, _IN)

    out = pl.pallas_call(
        _bn_body,
        grid=(_GRID,),
        in_specs=[row_spec, v_spec, v_spec],
        out_specs=row_spec,
        out_shape=jax.ShapeDtypeStruct((_N, _IN), f32),
    )(res, scale, shift)
    return out


# speedup vs baseline: 1.2432x; 1.2432x over previous
import functools

import jax
import jax.numpy as jnp
from jax import lax
from jax.experimental import pallas as pl
from jax.experimental.pallas import tpu as pltpu
from jax.experimental.pallas import tpu_sc as plsc

_N = 100000
_E = 3200000
_IN = 16
_HALF = 8

_TILES = 16
_KB = 8
_CE = _KB * 128
_EROWS = _E // 128
_ROWS = 25088
_PROWS = _ROWS - _EROWS
_ROWS_PER_TILE = _ROWS // _TILES
_CHUNKS = _ROWS_PER_TILE // _KB
_NPAD = 100096
_TSLICE = _NPAD // _TILES
_FSLICE = _N // _TILES

_MESH = plsc.VectorSubcoreMesh(core_axis_name="c", subcore_axis_name="s")
_PARAMS = pltpu.CompilerParams(use_tc_tiling_on_sc=False)


def _load_idx(esrc, edst, epad, srcv, dstv, isem, r0):
    @pl.when(r0 < _EROWS)
    def _():
        c1 = pltpu.async_copy(esrc.at[pl.ds(r0, _KB)], srcv, isem)
        c2 = pltpu.async_copy(edst.at[pl.ds(r0, _KB)], dstv, isem)
        c1.wait()
        c2.wait()

    @pl.when(r0 >= _EROWS)
    def _():
        c1 = pltpu.async_copy(epad.at[0, pl.ds(r0 - _EROWS, _KB)], srcv, isem)
        c2 = pltpu.async_copy(epad.at[1, pl.ds(r0 - _EROWS, _KB)], dstv, isem)
        c1.wait()
        c2.wait()


def _zero_slice(zeros_h, accum, s):
    pltpu.sync_copy(zeros_h.at[pl.ds(s * _TSLICE, _TSLICE)],
                    accum.at[pl.ds(s * _TSLICE, _TSLICE)])


def _writeout(accum, out, c, s):
    pltpu.sync_copy(accum.at[pl.ds(s * _TSLICE, _TSLICE)],
                    out.at[c, pl.ds(s * _TSLICE, _TSLICE)])


def _agg_body(feat_a, esrc, edst, epad, zeros_h,
              z1o, z2o, z4o, fa_tbl,
              accum, srcv, dstv, rows, fbuf, isem, gsem, ssem):
    c = lax.axis_index("c")
    s = lax.axis_index("s")
    rbase = s * _ROWS_PER_TILE

    for h in range(2):
        r = s * _FSLICE + h * (_FSLICE // 2)
        pltpu.sync_copy(
            feat_a.at[pl.ds(r, _FSLICE // 2), pl.ds(c * _HALF, _HALF)], fbuf)
        pltpu.sync_copy(fbuf, fa_tbl.at[c, pl.ds(r, _FSLICE // 2)])

    @pl.when(s == 0)
    def _():
        pltpu.sync_copy(zeros_h.at[pl.ds(_N, _NPAD - _N)],
                        fa_tbl.at[c, pl.ds(_N, _NPAD - _N)])

    def agg_pass(tbl):
        def body(i, carry):
            r0 = rbase + i * _KB
            _load_idx(esrc, edst, epad, srcv, dstv, isem, r0)
            gs = [pltpu.async_copy(tbl.at[srcv.at[j]],
                                   rows.at[pl.ds(j * 128, 128)], gsem)
                  for j in range(_KB)]
            for g in gs:
                g.wait()
            ss = [pltpu.async_copy(rows.at[pl.ds(j * 128, 128)],
                                   accum.at[dstv.at[j]], ssem, add=True)
                  for j in range(_KB)]
            for t in ss:
                t.wait()
            return carry
        lax.fori_loop(0, _CHUNKS, body, 0)

    _zero_slice(zeros_h, accum, s)
    plsc.subcore_barrier()
    agg_pass(fa_tbl.at[c])
    plsc.subcore_barrier()
    _writeout(accum, z1o, c, s)
    _zero_slice(zeros_h, accum, s)
    plsc.subcore_barrier()
    agg_pass(z1o.at[c])
    plsc.subcore_barrier()
    _writeout(accum, z2o, c, s)
    _zero_slice(zeros_h, accum, s)
    plsc.subcore_barrier()
    agg_pass(z2o.at[c])
    plsc.subcore_barrier()
    _writeout(accum, z4o, c, s)
    _zero_slice(zeros_h, accum, s)
    plsc.subcore_barrier()
    agg_pass(z4o.at[c])
    plsc.subcore_barrier()
    _writeout(accum, z4o, c, s)


def _pmpd_body(feat_b, esrc, edst, epad, zeros_h, pmo,
               accum, srcv, dstv, rows, isem, gsem, ssem):
    c = lax.axis_index("c")
    s = lax.axis_index("s")
    rbase = s * _ROWS_PER_TILE

    _zero_slice(zeros_h, accum, s)
    plsc.subcore_barrier()

    def body(i, carry):
        r0 = rbase + i * _KB
        _load_idx(esrc, edst, epad, srcv, dstv, isem, r0)

        @pl.when(r0 < _EROWS)
        def _():
            pltpu.async_copy(
                feat_b.at[pl.ds(r0 * 128, _CE), pl.ds(c * _HALF, _HALF)],
                rows, gsem).wait()

        ss = []
        for j in range(_KB):
            ss.append(pltpu.async_copy(rows.at[pl.ds(j * 128, 128)],
                                       accum.at[srcv.at[j]], ssem, add=True))
            ss.append(pltpu.async_copy(rows.at[pl.ds(j * 128, 128)],
                                       accum.at[dstv.at[j]], ssem, add=True))
        for t in ss:
            t.wait()
        return carry
    lax.fori_loop(0, _CHUNKS, body, 0)
    plsc.subcore_barrier()
    _writeout(accum, pmo, c, s)


_agg_call = functools.partial(
    pl.kernel,
    out_type=[jax.ShapeDtypeStruct((2, _NPAD, _HALF), jnp.float32)] * 4,
    mesh=_MESH,
    scratch_types=[
        pltpu.VMEM_SHARED((_NPAD, _HALF), jnp.float32),
        pltpu.VMEM((_KB, 128), jnp.int32),
        pltpu.VMEM((_KB, 128), jnp.int32),
        pltpu.VMEM((_CE, _HALF), jnp.float32),
        pltpu.VMEM((_FSLICE // 2, _HALF), jnp.float32),
        pltpu.SemaphoreType.DMA,
        pltpu.SemaphoreType.DMA,
        pltpu.SemaphoreType.DMA,
    ],
    compiler_params=_PARAMS,
)(_agg_body)

_pmpd_call = functools.partial(
    pl.kernel,
    out_type=jax.ShapeDtypeStruct((2, _NPAD, _HALF), jnp.float32),
    mesh=_MESH,
    scratch_types=[
        pltpu.VMEM_SHARED((_NPAD, _HALF), jnp.float32),
        pltpu.VMEM((_KB, 128), jnp.int32),
        pltpu.VMEM((_KB, 128), jnp.int32),
        pltpu.VMEM((_CE, _HALF), jnp.float32),
        pltpu.SemaphoreType.DMA,
        pltpu.SemaphoreType.DMA,
        pltpu.SemaphoreType.DMA,
    ],
    compiler_params=_PARAMS,
)(_pmpd_body)


_BN = 2000
_GRID = _N // _BN


def _fuse_body(fa, dg, z1, z2, z4, pm, wp, wd, wr, bsum, res, sums):
    x = fa[...]
    w = wr[...]
    acc = jnp.dot(x, wp[...], preferred_element_type=jnp.float32)
    acc += jnp.dot(dg[...] * x, wd[...], preferred_element_type=jnp.float32)
    acc += jnp.dot(z1[0], w[0], preferred_element_type=jnp.float32)
    acc += jnp.dot(z1[1], w[1], preferred_element_type=jnp.float32)
    acc += jnp.dot(z2[0], w[2], preferred_element_type=jnp.float32)
    acc += jnp.dot(z2[1], w[3], preferred_element_type=jnp.float32)
    acc += jnp.dot(z4[0], w[4], preferred_element_type=jnp.float32)
    acc += jnp.dot(z4[1], w[5], preferred_element_type=jnp.float32)
    acc += jnp.dot(pm[0], w[6], preferred_element_type=jnp.float32)
    acc += jnp.dot(pm[1], w[7], preferred_element_type=jnp.float32)
    acc += bsum[...]
    col = lax.broadcasted_iota(jnp.int32, acc.shape, 1)
    acc = jnp.where((col >= _IN // 2) & (acc < 0.0), 0.0, acc)
    res[...] = acc
    sums[...] = jnp.stack([jnp.sum(acc, axis=0),
                           jnp.sum(acc * acc, axis=0)])[None]


def _bn_body(res, scale, shift, out):
    out[...] = res[...] * scale[...] + shift[...]


def kernel(feat_a, feat_b, deg, edge_index, Wp, bp, Wd, bd, Wr0, br0,
           Wr1, br1, Wr2, br2, Wf, bf, gamma, beta):
    f32 = jnp.float32
    esrc = edge_index[0].reshape(_EROWS, 128)
    edst = edge_index[1].reshape(_EROWS, 128)
    epad = jnp.full((2, _PROWS, 128), _N, jnp.int32)
    zeros_h = jnp.zeros((_NPAD, _HALF), f32)

    z1h, z2h, z4h, _ = _agg_call(feat_a, esrc, edst, epad, zeros_h)
    pmh = _pmpd_call(feat_b, esrc, edst, epad, zeros_h)

    wr = jnp.stack([Wr0[:_HALF], Wr0[_HALF:], Wr1[:_HALF], Wr1[_HALF:],
                    Wr2[:_HALF], Wr2[_HALF:], Wf[:_HALF], Wf[_HALF:]])
    bsum = (bp + bd + br0 + br1 + br2 + bf).reshape(1, _IN)

    row_spec = pl.BlockSpec((_BN, _IN), lambda i: (i, 0))
    half_spec = pl.BlockSpec((2, _BN, _HALF), lambda i: (0, i, 0))
    deg_spec = pl.BlockSpec((_BN, 1), lambda i: (i, 0))
    w_spec = pl.BlockSpec((_IN, _IN), lambda i: (0, 0))
    wr_spec = pl.BlockSpec((8, _HALF, _IN), lambda i: (0, 0, 0))
    v_spec = pl.BlockSpec((1, _IN), lambda i: (0, 0))

    res, sums = pl.pallas_call(
        _fuse_body,
        grid=(_GRID,),
        in_specs=[row_spec, deg_spec, half_spec, half_spec, half_spec,
                  half_spec, w_spec, w_spec, wr_spec, v_spec],
        out_specs=[row_spec, pl.BlockSpec((1, 2, _IN), lambda i: (i, 0, 0))],
        out_shape=[jax.ShapeDtypeStruct((_N, _IN), f32),
                   jax.ShapeDtypeStruct((_GRID, 2, _IN), f32)],
    )(feat_a, deg, z1h, z2h, z4h, pmh, Wp, Wd, wr, bsum)

    st = jnp.sum(sums, axis=0)
    mean = st[0] / _N
    var = st[1] / _N - mean * mean
    rstd = lax.rsqrt(var + 1e-5)
    scale = (gamma * rstd).reshape(1, _IN)
    shift = (beta - mean * gamma * rstd).reshape(1, _IN)

    out = pl.pallas_call(
        _bn_body,
        grid=(_GRID,),
        in_specs=[row_spec, v_spec, v_spec],
        out_specs=row_spec,
        out_shape=jax.ShapeDtypeStruct((_N, _IN), f32),
    )(res, scale, shift)
    return out


# speedup vs baseline: 1.3615x; 1.0952x over previous
import functools

import jax
import jax.numpy as jnp
from jax import lax
from jax.experimental import pallas as pl
from jax.experimental.pallas import tpu as pltpu
from jax.experimental.pallas import tpu_sc as plsc

_N = 100000
_E = 3200000
_IN = 16
_HALF = 8

_TILES = 16
_KB = 8
_CE = _KB * 128
_EROWS = _E // 128
_ROWS = 25088
_PROWS = _ROWS - _EROWS
_ROWS_PER_TILE = _ROWS // _TILES
_CHUNKS = _ROWS_PER_TILE // _KB
_NPAD = 100096
_TSLICE = _NPAD // _TILES
_FSLICE = _N // _TILES

_MESH = plsc.VectorSubcoreMesh(core_axis_name="c", subcore_axis_name="s")
_PARAMS = pltpu.CompilerParams(use_tc_tiling_on_sc=False)


def _load_idx(esrc, edst, epad, srcv, dstv, isem, r0):
    @pl.when(r0 < _EROWS)
    def _():
        c1 = pltpu.async_copy(esrc.at[pl.ds(r0, _KB)], srcv, isem)
        c2 = pltpu.async_copy(edst.at[pl.ds(r0, _KB)], dstv, isem)
        c1.wait()
        c2.wait()

    @pl.when(r0 >= _EROWS)
    def _():
        c1 = pltpu.async_copy(epad.at[0, pl.ds(r0 - _EROWS, _KB)], srcv, isem)
        c2 = pltpu.async_copy(epad.at[1, pl.ds(r0 - _EROWS, _KB)], dstv, isem)
        c1.wait()
        c2.wait()


def _zero_slice(zeros_h, accum, s):
    pltpu.sync_copy(zeros_h.at[pl.ds(s * _TSLICE, _TSLICE)],
                    accum.at[pl.ds(s * _TSLICE, _TSLICE)])


def _writeout(accum, out, c, s):
    pltpu.sync_copy(accum.at[pl.ds(s * _TSLICE, _TSLICE)],
                    out.at[c, pl.ds(s * _TSLICE, _TSLICE)])


def _agg_body(feat_a, esrc, edst, epad, zeros_h,
              z1o, z2o, z4o, fa_tbl,
              accum, srcv0, srcv1, dstv0, dstv1, rows0, rows1, fbuf,
              isem, gsem0, gsem1, ssem0, ssem1):
    c = lax.axis_index("c")
    s = lax.axis_index("s")
    rbase = s * _ROWS_PER_TILE
    srcv, dstv = [srcv0, srcv1], [dstv0, dstv1]
    rows, gsem, ssem = [rows0, rows1], [gsem0, gsem1], [ssem0, ssem1]

    for h in range(2):
        r = s * _FSLICE + h * (_FSLICE // 2)
        pltpu.sync_copy(
            feat_a.at[pl.ds(r, _FSLICE // 2), pl.ds(c * _HALF, _HALF)], fbuf)
        pltpu.sync_copy(fbuf, fa_tbl.at[c, pl.ds(r, _FSLICE // 2)])

    @pl.when(s == 0)
    def _():
        pltpu.sync_copy(zeros_h.at[pl.ds(_N, _NPAD - _N)],
                        fa_tbl.at[c, pl.ds(_N, _NPAD - _N)])

    def agg_pass(tbl):
        def issue_g(k, b):
            _load_idx(esrc, edst, epad, srcv[b], dstv[b], isem,
                      rbase + k * _KB)
            for j in range(_KB):
                pltpu.async_copy(tbl.at[srcv[b].at[j]],
                                 rows[b].at[pl.ds(j * 128, 128)], gsem[b])

        def wait_g(b):
            for j in range(_KB):
                pltpu.make_async_copy(tbl.at[srcv[b].at[j]],
                                      rows[b].at[pl.ds(j * 128, 128)],
                                      gsem[b]).wait()

        def issue_s(b):
            for j in range(_KB):
                pltpu.async_copy(rows[b].at[pl.ds(j * 128, 128)],
                                 accum.at[dstv[b].at[j]], ssem[b], add=True)

        def drain_s(b):
            for j in range(_KB):
                pltpu.make_async_copy(rows[b].at[pl.ds(j * 128, 128)],
                                      accum.at[dstv[b].at[j]],
                                      ssem[b]).wait()

        def section(k, b, do_prev, do_drain):
            if do_prev:
                wait_g(1 - b)
                issue_s(1 - b)
            if do_drain:
                drain_s(b)
            issue_g(k, b)

        section(0, 0, False, False)
        section(1, 1, True, False)

        def body(i, carry):
            section(2 + 2 * i, 0, True, True)
            section(3 + 2 * i, 1, True, True)
            return carry
        lax.fori_loop(0, (_CHUNKS - 2) // 2, body, 0)

        wait_g(1)
        issue_s(1)
        drain_s(0)
        drain_s(1)

    _zero_slice(zeros_h, accum, s)
    plsc.subcore_barrier()
    agg_pass(fa_tbl.at[c])
    plsc.subcore_barrier()
    _writeout(accum, z1o, c, s)
    _zero_slice(zeros_h, accum, s)
    plsc.subcore_barrier()
    agg_pass(z1o.at[c])
    plsc.subcore_barrier()
    _writeout(accum, z2o, c, s)
    _zero_slice(zeros_h, accum, s)
    plsc.subcore_barrier()
    agg_pass(z2o.at[c])
    plsc.subcore_barrier()
    _writeout(accum, z4o, c, s)
    _zero_slice(zeros_h, accum, s)
    plsc.subcore_barrier()
    agg_pass(z4o.at[c])
    plsc.subcore_barrier()
    _writeout(accum, z4o, c, s)


def _pmpd_body(feat_b, esrc, edst, epad, zeros_h, pmo,
               accum, srcv0, srcv1, dstv0, dstv1, rows0, rows1,
               isem, gsem0, gsem1, ssem0, ssem1):
    c = lax.axis_index("c")
    s = lax.axis_index("s")
    rbase = s * _ROWS_PER_TILE
    srcv, dstv = [srcv0, srcv1], [dstv0, dstv1]
    rows, gsem, ssem = [rows0, rows1], [gsem0, gsem1], [ssem0, ssem1]

    _zero_slice(zeros_h, accum, s)
    plsc.subcore_barrier()

    def fb_copy(k, b):
        return pltpu.make_async_copy(
            feat_b.at[pl.ds((rbase + k * _KB) * 128, _CE),
                      pl.ds(c * _HALF, _HALF)],
            rows[b], gsem[b])

    def issue_g(k, b):
        _load_idx(esrc, edst, epad, srcv[b], dstv[b], isem, rbase + k * _KB)

        @pl.when(rbase + k * _KB < _EROWS)
        def _():
            fb_copy(k, b).start()

    def wait_g(k, b):
        @pl.when(rbase + k * _KB < _EROWS)
        def _():
            fb_copy(k, b).wait()

    def issue_s(b):
        for j in range(_KB):
            pltpu.async_copy(rows[b].at[pl.ds(j * 128, 128)],
                             accum.at[srcv[b].at[j]], ssem[b], add=True)
            pltpu.async_copy(rows[b].at[pl.ds(j * 128, 128)],
                             accum.at[dstv[b].at[j]], ssem[b], add=True)

    def drain_s(b):
        for j in range(_KB):
            pltpu.make_async_copy(rows[b].at[pl.ds(j * 128, 128)],
                                  accum.at[srcv[b].at[j]], ssem[b]).wait()
            pltpu.make_async_copy(rows[b].at[pl.ds(j * 128, 128)],
                                  accum.at[dstv[b].at[j]], ssem[b]).wait()

    def section(k, b, do_prev, do_drain):
        if do_prev:
            wait_g(k - 1, 1 - b)
            issue_s(1 - b)
        if do_drain:
            drain_s(b)
        issue_g(k, b)

    section(0, 0, False, False)
    section(1, 1, True, False)

    def body(i, carry):
        section(2 + 2 * i, 0, True, True)
        section(3 + 2 * i, 1, True, True)
        return carry
    lax.fori_loop(0, (_CHUNKS - 2) // 2, body, 0)

    wait_g(_CHUNKS - 1, 1)
    issue_s(1)
    drain_s(0)
    drain_s(1)

    plsc.subcore_barrier()
    _writeout(accum, pmo, c, s)


_agg_call = functools.partial(
    pl.kernel,
    out_type=[jax.ShapeDtypeStruct((2, _NPAD, _HALF), jnp.float32)] * 4,
    mesh=_MESH,
    scratch_types=[
        pltpu.VMEM_SHARED((_NPAD, _HALF), jnp.float32),
        pltpu.VMEM((_KB, 128), jnp.int32),
        pltpu.VMEM((_KB, 128), jnp.int32),
        pltpu.VMEM((_KB, 128), jnp.int32),
        pltpu.VMEM((_KB, 128), jnp.int32),
        pltpu.VMEM((_CE, _HALF), jnp.float32),
        pltpu.VMEM((_CE, _HALF), jnp.float32),
        pltpu.VMEM((_FSLICE // 2, _HALF), jnp.float32),
        pltpu.SemaphoreType.DMA,
        pltpu.SemaphoreType.DMA,
        pltpu.SemaphoreType.DMA,
        pltpu.SemaphoreType.DMA,
        pltpu.SemaphoreType.DMA,
    ],
    compiler_params=_PARAMS,
)(_agg_body)

_pmpd_call = functools.partial(
    pl.kernel,
    out_type=jax.ShapeDtypeStruct((2, _NPAD, _HALF), jnp.float32),
    mesh=_MESH,
    scratch_types=[
        pltpu.VMEM_SHARED((_NPAD, _HALF), jnp.float32),
        pltpu.VMEM((_KB, 128), jnp.int32),
        pltpu.VMEM((_KB, 128), jnp.int32),
        pltpu.VMEM((_KB, 128), jnp.int32),
        pltpu.VMEM((_KB, 128), jnp.int32),
        pltpu.VMEM((_CE, _HALF), jnp.float32),
        pltpu.VMEM((_CE, _HALF), jnp.float32),
        pltpu.SemaphoreType.DMA,
        pltpu.SemaphoreType.DMA,
        pltpu.SemaphoreType.DMA,
        pltpu.SemaphoreType.DMA,
        pltpu.SemaphoreType.DMA,
    ],
    compiler_params=_PARAMS,
)(_pmpd_body)


_BN = 2000
_GRID = _N // _BN


def _fuse_body(fa, dg, z1, z2, z4, pm, wp, wd, wr, bsum, res, sums):
    x = fa[...]
    w = wr[...]
    acc = jnp.dot(x, wp[...], preferred_element_type=jnp.float32)
    acc += jnp.dot(dg[...] * x, wd[...], preferred_element_type=jnp.float32)
    acc += jnp.dot(z1[0], w[0], preferred_element_type=jnp.float32)
    acc += jnp.dot(z1[1], w[1], preferred_element_type=jnp.float32)
    acc += jnp.dot(z2[0], w[2], preferred_element_type=jnp.float32)
    acc += jnp.dot(z2[1], w[3], preferred_element_type=jnp.float32)
    acc += jnp.dot(z4[0], w[4], preferred_element_type=jnp.float32)
    acc += jnp.dot(z4[1], w[5], preferred_element_type=jnp.float32)
    acc += jnp.dot(pm[0], w[6], preferred_element_type=jnp.float32)
    acc += jnp.dot(pm[1], w[7], preferred_element_type=jnp.float32)
    acc += bsum[...]
    col = lax.broadcasted_iota(jnp.int32, acc.shape, 1)
    acc = jnp.where((col >= _IN // 2) & (acc < 0.0), 0.0, acc)
    res[...] = acc
    sums[...] = jnp.stack([jnp.sum(acc, axis=0),
                           jnp.sum(acc * acc, axis=0)])[None]


def _bn_body(res, scale, shift, out):
    out[...] = res[...] * scale[...] + shift[...]


def kernel(feat_a, feat_b, deg, edge_index, Wp, bp, Wd, bd, Wr0, br0,
           Wr1, br1, Wr2, br2, Wf, bf, gamma, beta):
    f32 = jnp.float32
    esrc = edge_index[0].reshape(_EROWS, 128)
    edst = edge_index[1].reshape(_EROWS, 128)
    epad = jnp.full((2, _PROWS, 128), _N, jnp.int32)
    zeros_h = jnp.zeros((_NPAD, _HALF), f32)

    z1h, z2h, z4h, _ = _agg_call(feat_a, esrc, edst, epad, zeros_h)
    pmh = _pmpd_call(feat_b, esrc, edst, epad, zeros_h)

    wr = jnp.stack([Wr0[:_HALF], Wr0[_HALF:], Wr1[:_HALF], Wr1[_HALF:],
                    Wr2[:_HALF], Wr2[_HALF:], Wf[:_HALF], Wf[_HALF:]])
    bsum = (bp + bd + br0 + br1 + br2 + bf).reshape(1, _IN)

    row_spec = pl.BlockSpec((_BN, _IN), lambda i: (i, 0))
    half_spec = pl.BlockSpec((2, _BN, _HALF), lambda i: (0, i, 0))
    deg_spec = pl.BlockSpec((_BN, 1), lambda i: (i, 0))
    w_spec = pl.BlockSpec((_IN, _IN), lambda i: (0, 0))
    wr_spec = pl.BlockSpec((8, _HALF, _IN), lambda i: (0, 0, 0))
    v_spec = pl.BlockSpec((1, _IN), lambda i: (0, 0))

    res, sums = pl.pallas_call(
        _fuse_body,
        grid=(_GRID,),
        in_specs=[row_spec, deg_spec, half_spec, half_spec, half_spec,
                  half_spec, w_spec, w_spec, wr_spec, v_spec],
        out_specs=[row_spec, pl.BlockSpec((1, 2, _IN), lambda i: (i, 0, 0))],
        out_shape=[jax.ShapeDtypeStruct((_N, _IN), f32),
                   jax.ShapeDtypeStruct((_GRID, 2, _IN), f32)],
    )(feat_a, deg, z1h, z2h, z4h, pmh, Wp, Wd, wr, bsum)

    st = jnp.sum(sums, axis=0)
    mean = st[0] / _N
    var = st[1] / _N - mean * mean
    rstd = lax.rsqrt(var + 1e-5)
    scale = (gamma * rstd).reshape(1, _IN)
    shift = (beta - mean * gamma * rstd).reshape(1, _IN)

    out = pl.pallas_call(
        _bn_body,
        grid=(_GRID,),
        in_specs=[row_spec, v_spec, v_spec],
        out_specs=row_spec,
        out_shape=jax.ShapeDtypeStruct((_N, _IN), f32),
    )(res, scale, shift)
    return out


# speedup vs baseline: 1.3702x; 1.0064x over previous
import functools

import jax
import jax.numpy as jnp
from jax import lax
from jax.experimental import pallas as pl
from jax.experimental.pallas import tpu as pltpu
from jax.experimental.pallas import tpu_sc as plsc

_N = 100000
_E = 3200000
_IN = 16
_HALF = 8

_TILES = 16
_KB = 8
_CE = _KB * 128
_EROWS = _E // 128
_ROWS = 25088
_PROWS = _ROWS - _EROWS
_ROWS_PER_TILE = _ROWS // _TILES
_CHUNKS = _ROWS_PER_TILE // _KB
_NPAD = 100096
_TSLICE = _NPAD // _TILES
_FSLICE = _N // _TILES

_MESH = plsc.VectorSubcoreMesh(core_axis_name="c", subcore_axis_name="s")
_PARAMS = pltpu.CompilerParams(use_tc_tiling_on_sc=False)


def _load_idx(esrc, edst, epad, srcv, dstv, isem, r0):
    @pl.when(r0 < _EROWS)
    def _():
        c1 = pltpu.async_copy(esrc.at[pl.ds(r0 * 128, _CE)], srcv, isem)
        c2 = pltpu.async_copy(edst.at[pl.ds(r0 * 128, _CE)], dstv, isem)
        c1.wait()
        c2.wait()

    @pl.when(r0 >= _EROWS)
    def _():
        c1 = pltpu.async_copy(epad.at[0, pl.ds((r0 - _EROWS) * 128, _CE)],
                              srcv, isem)
        c2 = pltpu.async_copy(epad.at[1, pl.ds((r0 - _EROWS) * 128, _CE)],
                              dstv, isem)
        c1.wait()
        c2.wait()


def _zero_slice(zeros_h, accum, s):
    pltpu.sync_copy(zeros_h.at[pl.ds(s * _TSLICE, _TSLICE)],
                    accum.at[pl.ds(s * _TSLICE, _TSLICE)])


def _writeout(accum, out, c, s):
    pltpu.sync_copy(accum.at[pl.ds(s * _TSLICE, _TSLICE)],
                    out.at[c, pl.ds(s * _TSLICE, _TSLICE)])


def _agg_body(feat_a, esrc, edst, epad, zeros_h,
              z1o, z2o, z4o, fa_tbl,
              accum, srcv0, srcv1, dstv0, dstv1, rows0, rows1, fbuf,
              isem, gsem0, gsem1, ssem0, ssem1):
    c = lax.axis_index("c")
    s = lax.axis_index("s")
    rbase = s * _ROWS_PER_TILE
    srcv, dstv = [srcv0, srcv1], [dstv0, dstv1]
    rows, gsem, ssem = [rows0, rows1], [gsem0, gsem1], [ssem0, ssem1]

    for h in range(2):
        r = s * _FSLICE + h * (_FSLICE // 2)
        pltpu.sync_copy(
            feat_a.at[pl.ds(r, _FSLICE // 2), pl.ds(c * _HALF, _HALF)], fbuf)
        pltpu.sync_copy(fbuf, fa_tbl.at[c, pl.ds(r, _FSLICE // 2)])

    @pl.when(s == 0)
    def _():
        pltpu.sync_copy(zeros_h.at[pl.ds(_N, _NPAD - _N)],
                        fa_tbl.at[c, pl.ds(_N, _NPAD - _N)])

    def agg_pass(tbl):
        def issue_g(k, b):
            _load_idx(esrc, edst, epad, srcv[b], dstv[b], isem,
                      rbase + k * _KB)
            pltpu.async_copy(tbl.at[srcv[b]], rows[b], gsem[b])

        def wait_g(b):
            pltpu.make_async_copy(tbl.at[srcv[b]], rows[b], gsem[b]).wait()

        def issue_s(b):
            pltpu.async_copy(rows[b], accum.at[dstv[b]], ssem[b], add=True)

        def drain_s(b):
            pltpu.make_async_copy(rows[b], accum.at[dstv[b]],
                                  ssem[b]).wait()

        def section(k, b, do_prev, do_drain):
            if do_prev:
                wait_g(1 - b)
                issue_s(1 - b)
            if do_drain:
                drain_s(b)
            issue_g(k, b)

        section(0, 0, False, False)
        section(1, 1, True, False)

        def body(i, carry):
            section(2 + 2 * i, 0, True, True)
            section(3 + 2 * i, 1, True, True)
            return carry
        lax.fori_loop(0, (_CHUNKS - 2) // 2, body, 0)

        wait_g(1)
        issue_s(1)
        drain_s(0)
        drain_s(1)

    _zero_slice(zeros_h, accum, s)
    plsc.subcore_barrier()
    agg_pass(fa_tbl.at[c])
    plsc.subcore_barrier()
    _writeout(accum, z1o, c, s)
    _zero_slice(zeros_h, accum, s)
    plsc.subcore_barrier()
    agg_pass(z1o.at[c])
    plsc.subcore_barrier()
    _writeout(accum, z2o, c, s)
    _zero_slice(zeros_h, accum, s)
    plsc.subcore_barrier()
    agg_pass(z2o.at[c])
    plsc.subcore_barrier()
    _writeout(accum, z4o, c, s)
    _zero_slice(zeros_h, accum, s)
    plsc.subcore_barrier()
    agg_pass(z4o.at[c])
    plsc.subcore_barrier()
    _writeout(accum, z4o, c, s)


def _pmpd_body(feat_b, esrc, edst, epad, zeros_h, pmo,
               accum, srcv0, srcv1, dstv0, dstv1, rows0, rows1,
               isem, gsem0, gsem1, ssem0, ssem1):
    c = lax.axis_index("c")
    s = lax.axis_index("s")
    rbase = s * _ROWS_PER_TILE
    srcv, dstv = [srcv0, srcv1], [dstv0, dstv1]
    rows, gsem, ssem = [rows0, rows1], [gsem0, gsem1], [ssem0, ssem1]

    _zero_slice(zeros_h, accum, s)
    plsc.subcore_barrier()

    def fb_copy(k, b):
        return pltpu.make_async_copy(
            feat_b.at[pl.ds((rbase + k * _KB) * 128, _CE),
                      pl.ds(c * _HALF, _HALF)],
            rows[b], gsem[b])

    def issue_g(k, b):
        _load_idx(esrc, edst, epad, srcv[b], dstv[b], isem, rbase + k * _KB)

        @pl.when(rbase + k * _KB < _EROWS)
        def _():
            fb_copy(k, b).start()

    def wait_g(k, b):
        @pl.when(rbase + k * _KB < _EROWS)
        def _():
            fb_copy(k, b).wait()

    def issue_s(b):
        pltpu.async_copy(rows[b], accum.at[srcv[b]], ssem[b], add=True)
        pltpu.async_copy(rows[b], accum.at[dstv[b]], ssem[b], add=True)

    def drain_s(b):
        pltpu.make_async_copy(rows[b], accum.at[srcv[b]], ssem[b]).wait()
        pltpu.make_async_copy(rows[b], accum.at[dstv[b]], ssem[b]).wait()

    def section(k, b, do_prev, do_drain):
        if do_prev:
            wait_g(k - 1, 1 - b)
            issue_s(1 - b)
        if do_drain:
            drain_s(b)
        issue_g(k, b)

    section(0, 0, False, False)
    section(1, 1, True, False)

    def body(i, carry):
        section(2 + 2 * i, 0, True, True)
        section(3 + 2 * i, 1, True, True)
        return carry
    lax.fori_loop(0, (_CHUNKS - 2) // 2, body, 0)

    wait_g(_CHUNKS - 1, 1)
    issue_s(1)
    drain_s(0)
    drain_s(1)

    plsc.subcore_barrier()
    _writeout(accum, pmo, c, s)


_agg_call = functools.partial(
    pl.kernel,
    out_type=[jax.ShapeDtypeStruct((2, _NPAD, _HALF), jnp.float32)] * 4,
    mesh=_MESH,
    scratch_types=[
        pltpu.VMEM_SHARED((_NPAD, _HALF), jnp.float32),
        pltpu.VMEM((_CE,), jnp.int32),
        pltpu.VMEM((_CE,), jnp.int32),
        pltpu.VMEM((_CE,), jnp.int32),
        pltpu.VMEM((_CE,), jnp.int32),
        pltpu.VMEM((_CE, _HALF), jnp.float32),
        pltpu.VMEM((_CE, _HALF), jnp.float32),
        pltpu.VMEM((_FSLICE // 2, _HALF), jnp.float32),
        pltpu.SemaphoreType.DMA,
        pltpu.SemaphoreType.DMA,
        pltpu.SemaphoreType.DMA,
        pltpu.SemaphoreType.DMA,
        pltpu.SemaphoreType.DMA,
    ],
    compiler_params=_PARAMS,
)(_agg_body)

_pmpd_call = functools.partial(
    pl.kernel,
    out_type=jax.ShapeDtypeStruct((2, _NPAD, _HALF), jnp.float32),
    mesh=_MESH,
    scratch_types=[
        pltpu.VMEM_SHARED((_NPAD, _HALF), jnp.float32),
        pltpu.VMEM((_CE,), jnp.int32),
        pltpu.VMEM((_CE,), jnp.int32),
        pltpu.VMEM((_CE,), jnp.int32),
        pltpu.VMEM((_CE,), jnp.int32),
        pltpu.VMEM((_CE, _HALF), jnp.float32),
        pltpu.VMEM((_CE, _HALF), jnp.float32),
        pltpu.SemaphoreType.DMA,
        pltpu.SemaphoreType.DMA,
        pltpu.SemaphoreType.DMA,
        pltpu.SemaphoreType.DMA,
        pltpu.SemaphoreType.DMA,
    ],
    compiler_params=_PARAMS,
)(_pmpd_body)


_BN = 2000
_GRID = _N // _BN


def _fuse_body(fa, dg, z1, z2, z4, pm, wp, wd, wr, bsum, res, sums):
    x = fa[...]
    w = wr[...]
    acc = jnp.dot(x, wp[...], preferred_element_type=jnp.float32)
    acc += jnp.dot(dg[...] * x, wd[...], preferred_element_type=jnp.float32)
    acc += jnp.dot(z1[0], w[0], preferred_element_type=jnp.float32)
    acc += jnp.dot(z1[1], w[1], preferred_element_type=jnp.float32)
    acc += jnp.dot(z2[0], w[2], preferred_element_type=jnp.float32)
    acc += jnp.dot(z2[1], w[3], preferred_element_type=jnp.float32)
    acc += jnp.dot(z4[0], w[4], preferred_element_type=jnp.float32)
    acc += jnp.dot(z4[1], w[5], preferred_element_type=jnp.float32)
    acc += jnp.dot(pm[0], w[6], preferred_element_type=jnp.float32)
    acc += jnp.dot(pm[1], w[7], preferred_element_type=jnp.float32)
    acc += bsum[...]
    col = lax.broadcasted_iota(jnp.int32, acc.shape, 1)
    acc = jnp.where((col >= _IN // 2) & (acc < 0.0), 0.0, acc)
    res[...] = acc
    sums[...] = jnp.stack([jnp.sum(acc, axis=0),
                           jnp.sum(acc * acc, axis=0)])[None]


def _bn_body(res, scale, shift, out):
    out[...] = res[...] * scale[...] + shift[...]


def kernel(feat_a, feat_b, deg, edge_index, Wp, bp, Wd, bd, Wr0, br0,
           Wr1, br1, Wr2, br2, Wf, bf, gamma, beta):
    f32 = jnp.float32
    esrc = edge_index[0]
    edst = edge_index[1]
    epad = jnp.full((2, _PROWS * 128), _N, jnp.int32)
    zeros_h = jnp.zeros((_NPAD, _HALF), f32)

    z1h, z2h, z4h, _ = _agg_call(feat_a, esrc, edst, epad, zeros_h)
    pmh = _pmpd_call(feat_b, esrc, edst, epad, zeros_h)

    wr = jnp.stack([Wr0[:_HALF], Wr0[_HALF:], Wr1[:_HALF], Wr1[_HALF:],
                    Wr2[:_HALF], Wr2[_HALF:], Wf[:_HALF], Wf[_HALF:]])
    bsum = (bp + bd + br0 + br1 + br2 + bf).reshape(1, _IN)

    row_spec = pl.BlockSpec((_BN, _IN), lambda i: (i, 0))
    half_spec = pl.BlockSpec((2, _BN, _HALF), lambda i: (0, i, 0))
    deg_spec = pl.BlockSpec((_BN, 1), lambda i: (i, 0))
    w_spec = pl.BlockSpec((_IN, _IN), lambda i: (0, 0))
    wr_spec = pl.BlockSpec((8, _HALF, _IN), lambda i: (0, 0, 0))
    v_spec = pl.BlockSpec((1, _IN), lambda i: (0, 0))

    res, sums = pl.pallas_call(
        _fuse_body,
        grid=(_GRID,),
        in_specs=[row_spec, deg_spec, half_spec, half_spec, half_spec,
                  half_spec, w_spec, w_spec, wr_spec, v_spec],
        out_specs=[row_spec, pl.BlockSpec((1, 2, _IN), lambda i: (i, 0, 0))],
        out_shape=[jax.ShapeDtypeStruct((_N, _IN), f32),
                   jax.ShapeDtypeStruct((_GRID, 2, _IN), f32)],
    )(feat_a, deg, z1h, z2h, z4h, pmh, Wp, Wd, wr, bsum)

    st = jnp.sum(sums, axis=0)
    mean = st[0] / _N
    var = st[1] / _N - mean * mean
    rstd = lax.rsqrt(var + 1e-5)
    scale = (gamma * rstd).reshape(1, _IN)
    shift = (beta - mean * gamma * rstd).reshape(1, _IN)

    out = pl.pallas_call(
        _bn_body,
        grid=(_GRID,),
        in_specs=[row_spec, v_spec, v_spec],
        out_specs=row_spec,
        out_shape=jax.ShapeDtypeStruct((_N, _IN), f32),
    )(res, scale, shift)
    return out
